# Initial kernel scaffold; baseline (speedup 1.0000x reference)
import jax
import jax.numpy as jnp
from jax.experimental import pallas as pl


def kernel(x, edge_index, batch, W1, b1, g1, be1, W2, b2, g2, be2, Wfc, bfc):
    raise NotImplementedError("write your pallas kernel here")


# speedup vs baseline: 26.5023x; 26.5023x over previous
import functools

import jax
import jax.numpy as jnp
from jax import lax
from jax.experimental import pallas as pl
from jax.experimental.pallas import tpu as pltpu
from jax.experimental.pallas import tpu_sc as plsc

_N = 10000
_E = 320000
_D = 128
_H = 128
_OUT = 64
_G = 64
_EPS = 1e-5

_NC = 2
_NS = 16
_HH = _H // _NC
_K = 128
_CH2 = 160
_CH = _CH2 // _NC
_EP = _NS * _CH2 * _K
_NPAD = 10112
_SEG = _NPAD // _NS


def _sc_mesh():
    return plsc.VectorSubcoreMesh(
        core_axis_name="c", subcore_axis_name="s",
        num_cores=_NC, num_subcores=_NS)


@functools.partial(
    pl.kernel,
    out_type=jax.ShapeDtypeStruct((_NC * _NPAD,), jnp.float32),
    mesh=_sc_mesh(),
    scratch_types=[
        pltpu.VMEM((_CH, _K), jnp.int32),
        pltpu.VMEM((_K,), jnp.float32),
        pltpu.VMEM((640,), jnp.float32),
        pltpu.VMEM_SHARED((_NPAD,), jnp.float32),
        pltpu.SemaphoreType.DMA,
    ],
)
def _sc_degree(c_hbm, out_hbm, idx_v, ones_v, zbuf, deg_sh, sem):
    cid = lax.axis_index("c")
    sid = lax.axis_index("s")

    @pl.loop(0, 40)
    def _fill_z(i):
        zbuf[pl.ds(pl.multiple_of(i * 16, 16), 16)] = jnp.zeros((16,), jnp.float32)

    @pl.loop(0, 8)
    def _fill_1(i):
        ones_v[pl.ds(pl.multiple_of(i * 16, 16), 16)] = jnp.ones((16,), jnp.float32)

    off = pl.multiple_of(sid * _SEG, 8)
    pltpu.sync_copy(zbuf.at[pl.ds(0, _SEG)], deg_sh.at[pl.ds(off, _SEG)])
    pltpu.sync_copy(c_hbm.at[sid, pl.ds(pl.multiple_of(cid * _CH, 8), _CH)],
                    idx_v)
    plsc.subcore_barrier()

    @pl.loop(0, _CH, step=8)
    def _chunks(j):
        descs = [
            pltpu.async_copy(ones_v, deg_sh.at[idx_v.at[j + b]], sem, add=True)
            for b in range(8)
        ]
        for d in descs:
            d.wait()

    plsc.subcore_barrier()
    out_off = pl.multiple_of(cid * _NPAD + sid * _SEG, 8)
    pltpu.sync_copy(deg_sh.at[pl.ds(off, _SEG)], zbuf.at[pl.ds(0, _SEG)])
    pltpu.sync_copy(zbuf.at[pl.ds(0, _SEG)], out_hbm.at[pl.ds(out_off, _SEG)])


@functools.partial(
    pl.kernel,
    out_type=jax.ShapeDtypeStruct((_NC, _NPAD, _HH), jnp.float32),
    mesh=_sc_mesh(),
    scratch_types=[
        pltpu.VMEM((_CH2, _K), jnp.int32),
        pltpu.VMEM((_CH2, _K), jnp.int32),
        pltpu.VMEM((_K, _HH), jnp.float32),
        pltpu.VMEM((_K, _HH), jnp.float32),
        pltpu.VMEM_SHARED((_NPAD, _HH), jnp.float32),
        pltpu.SemaphoreType.DMA,
        pltpu.SemaphoreType.DMA,
    ],
    compiler_params=pltpu.CompilerParams(use_tc_tiling_on_sc=False),
)
def _sc_msg(y_hbm, r_hbm, c_hbm, out_hbm,
            r_v, c_v, buf0, buf1, agg_sh, sem0, sem1):
    cid = lax.axis_index("c")
    sid = lax.axis_index("s")
    base = pl.multiple_of(sid * _SEG, 8)
    ytab = y_hbm.at[cid]

    @pl.loop(0, _K)
    def _fill_z(i):
        for t in range(_HH // 16):
            buf0[i, pl.ds(t * 16, 16)] = jnp.zeros((16,), jnp.float32)

    for t in range(4):
        pltpu.sync_copy(buf0, agg_sh.at[pl.ds(base + t * _K, _K)])
    pltpu.sync_copy(buf0.at[pl.ds(0, _SEG - 4 * _K)],
                    agg_sh.at[pl.ds(base + 4 * _K, _SEG - 4 * _K)])
    pltpu.sync_copy(r_hbm.at[sid], r_v)
    pltpu.sync_copy(c_hbm.at[sid], c_v)
    plsc.subcore_barrier()

    bufs = (buf0, buf1)
    sems = (sem0, sem1)
    pltpu.async_copy(ytab.at[r_v.at[0]], buf0, sem0)
    pltpu.async_copy(ytab.at[r_v.at[1]], buf1, sem1)

    @pl.loop(0, _CH2 - 2, step=2)
    def _chunks(j):
        for b in range(2):
            jj = j + b
            pltpu.make_async_copy(ytab.at[r_v.at[jj]], bufs[b], sems[b]).wait()
            pltpu.sync_copy(bufs[b], agg_sh.at[c_v.at[jj]], add=True)
            pltpu.async_copy(ytab.at[r_v.at[jj + 2]], bufs[b], sems[b])

    for b in range(2):
        jj = _CH2 - 2 + b
        pltpu.make_async_copy(ytab.at[r_v.at[jj]], bufs[b], sems[b]).wait()
        pltpu.sync_copy(bufs[b], agg_sh.at[c_v.at[jj]], add=True)

    plsc.subcore_barrier()
    tail = _SEG - 4 * _K
    for t in range(4):
        pltpu.sync_copy(agg_sh.at[pl.ds(base + t * _K, _K)], buf0)
        pltpu.sync_copy(buf0, out_hbm.at[cid, pl.ds(base + t * _K, _K)])
    pltpu.sync_copy(agg_sh.at[pl.ds(base + 4 * _K, tail)],
                    buf0.at[pl.ds(0, tail)])
    pltpu.sync_copy(buf0.at[pl.ds(0, tail)],
                    out_hbm.at[cid, pl.ds(base + 4 * _K, tail)])


def _tc_pre(x, W1, deg3):
    def body(x_ref, w_ref, deg_ref, xw_ref, y_ref, dinv_ref):
        deg = deg_ref[0] + deg_ref[1] + 1.0
        dinv = lax.rsqrt(deg)
        xw = jnp.dot(x_ref[...], w_ref[...], preferred_element_type=jnp.float32,
                     precision=lax.Precision.HIGHEST)
        xw_ref[...] = xw
        y = xw * dinv
        y_ref[0] = y[:, :_HH]
        y_ref[1] = y[:, _HH:]
        dinv_ref[...] = dinv

    return pl.pallas_call(
        body,
        out_shape=[
            jax.ShapeDtypeStruct((_N, _H), jnp.float32),
            jax.ShapeDtypeStruct((_NC, _N, _HH), jnp.float32),
            jax.ShapeDtypeStruct((_N, 1), jnp.float32),
        ],
        compiler_params=pltpu.CompilerParams(
            vmem_limit_bytes=100 * 1024 * 1024),
    )(x, W1, deg3)


def _tc_mid(agg, xw, dinv, b, g, be, W2):
    def body(a_ref, xw_ref, dinv_ref, b_ref, g_ref, be_ref, w_ref,
             xw2_ref, y2_ref):
        dinv = dinv_ref[...]
        edge = jnp.concatenate([a_ref[0, :_N], a_ref[1, :_N]], axis=1)
        t = edge * dinv + xw_ref[...] * (dinv * dinv) + b_ref[...]
        mu = jnp.mean(t, axis=0, keepdims=True)
        var = jnp.mean((t - mu) ** 2, axis=0, keepdims=True)
        h = jnp.maximum((t - mu) * lax.rsqrt(var + _EPS) * g_ref[...]
                        + be_ref[...], 0.0)
        xw2 = jnp.dot(h, w_ref[...], preferred_element_type=jnp.float32,
                     precision=lax.Precision.HIGHEST)
        xw2_ref[...] = xw2
        y2 = xw2 * dinv
        y2_ref[0] = y2[:, :_HH]
        y2_ref[1] = y2[:, _HH:]

    return pl.pallas_call(
        body,
        out_shape=[
            jax.ShapeDtypeStruct((_N, _H), jnp.float32),
            jax.ShapeDtypeStruct((_NC, _N, _HH), jnp.float32),
        ],
        compiler_params=pltpu.CompilerParams(
            vmem_limit_bytes=100 * 1024 * 1024),
    )(agg, xw, dinv, b, g, be, W2)


def _tc_post(agg, xw, dinv, b, g, be, batch_row, Wfc, bfc):
    def body(a_ref, xw_ref, dinv_ref, b_ref, g_ref, be_ref, batch_ref,
             wfc_ref, bfc_ref, out_ref):
        dinv = dinv_ref[...]
        edge = jnp.concatenate([a_ref[0, :_N], a_ref[1, :_N]], axis=1)
        t = edge * dinv + xw_ref[...] * (dinv * dinv) + b_ref[...]
        mu = jnp.mean(t, axis=0, keepdims=True)
        var = jnp.mean((t - mu) ** 2, axis=0, keepdims=True)
        h = jnp.maximum((t - mu) * lax.rsqrt(var + _EPS) * g_ref[...]
                        + be_ref[...], 0.0)
        gi = lax.broadcasted_iota(jnp.int32, (_G, _N), 0)
        oh = (gi == batch_ref[...]).astype(jnp.float32)
        sums = jnp.dot(oh, h, preferred_element_type=jnp.float32,
                     precision=lax.Precision.HIGHEST)
        cnt = jnp.sum(oh, axis=1, keepdims=True)
        pooled = sums / jnp.maximum(cnt, 1.0)
        out_ref[...] = (jnp.dot(pooled, wfc_ref[...],
                                preferred_element_type=jnp.float32,
                     precision=lax.Precision.HIGHEST)
                        + bfc_ref[...])

    return pl.pallas_call(
        body,
        out_shape=jax.ShapeDtypeStruct((_G, _OUT), jnp.float32),
        compiler_params=pltpu.CompilerParams(
            vmem_limit_bytes=100 * 1024 * 1024),
    )(agg, xw, dinv, b, g, be, batch_row, Wfc, bfc)


def kernel(x, edge_index, batch, W1, b1, g1, be1, W2, b2, g2, be2, Wfc, bfc):
    r = edge_index[0]
    c = edge_index[1]
    pad = jnp.arange(_EP - _E, dtype=jnp.int32)
    r3 = jnp.concatenate([r, pad % _N]).reshape(_NS, _CH2, _K)
    c3 = jnp.concatenate(
        [c, _N + pad % (_NPAD - _N)]).reshape(_NS, _CH2, _K)

    deg_parts = _sc_degree(c3)
    deg3 = deg_parts.reshape(_NC, _NPAD)[:, :_N].reshape(_NC, _N, 1)
    xw1, y1s, dinv = _tc_pre(x, W1, deg3)
    agg1 = _sc_msg(y1s, r3, c3)
    xw2, y2s = _tc_mid(agg1, xw1, dinv,
                       b1.reshape(1, _H), g1.reshape(1, _H), be1.reshape(1, _H),
                       W2)
    agg2 = _sc_msg(y2s, r3, c3)
    out = _tc_post(agg2, xw2, dinv,
                   b2.reshape(1, _H), g2.reshape(1, _H), be2.reshape(1, _H),
                   batch.reshape(1, _N), Wfc, bfc.reshape(1, _OUT))
    return out


# speedup vs baseline: 27.5593x; 1.0399x over previous
import functools

import jax
import jax.numpy as jnp
from jax import lax
from jax.experimental import pallas as pl
from jax.experimental.pallas import tpu as pltpu
from jax.experimental.pallas import tpu_sc as plsc

_N = 10000
_E = 320000
_D = 128
_H = 128
_OUT = 64
_G = 64
_EPS = 1e-5

_NC = 2
_NS = 16
_HH = _H // _NC
_K = 128
_CH2 = 160
_CH = _CH2 // _NC
_EP = _NS * _CH2 * _K
_NB = 4
_NPAD = 10112
_SEG = _NPAD // _NS


def _sc_mesh():
    return plsc.VectorSubcoreMesh(
        core_axis_name="c", subcore_axis_name="s",
        num_cores=_NC, num_subcores=_NS)


@functools.partial(
    pl.kernel,
    out_type=jax.ShapeDtypeStruct((_NC * _NPAD,), jnp.float32),
    mesh=_sc_mesh(),
    scratch_types=[
        pltpu.VMEM((_CH, _K), jnp.int32),
        pltpu.VMEM((_K,), jnp.float32),
        pltpu.VMEM((640,), jnp.float32),
        pltpu.VMEM_SHARED((_NPAD,), jnp.float32),
        pltpu.SemaphoreType.DMA,
    ],
)
def _sc_degree(c_hbm, out_hbm, idx_v, ones_v, zbuf, deg_sh, sem):
    cid = lax.axis_index("c")
    sid = lax.axis_index("s")

    @pl.loop(0, 40)
    def _fill_z(i):
        zbuf[pl.ds(pl.multiple_of(i * 16, 16), 16)] = jnp.zeros((16,), jnp.float32)

    @pl.loop(0, 8)
    def _fill_1(i):
        ones_v[pl.ds(pl.multiple_of(i * 16, 16), 16)] = jnp.ones((16,), jnp.float32)

    off = pl.multiple_of(sid * _SEG, 8)
    pltpu.sync_copy(zbuf.at[pl.ds(0, _SEG)], deg_sh.at[pl.ds(off, _SEG)])
    pltpu.sync_copy(c_hbm.at[sid, pl.ds(pl.multiple_of(cid * _CH, 8), _CH)],
                    idx_v)
    plsc.subcore_barrier()

    @pl.loop(0, _CH, step=8)
    def _chunks(j):
        descs = [
            pltpu.async_copy(ones_v, deg_sh.at[idx_v.at[j + b]], sem, add=True)
            for b in range(8)
        ]
        for d in descs:
            d.wait()

    plsc.subcore_barrier()
    out_off = pl.multiple_of(cid * _NPAD + sid * _SEG, 8)
    pltpu.sync_copy(deg_sh.at[pl.ds(off, _SEG)], zbuf.at[pl.ds(0, _SEG)])
    pltpu.sync_copy(zbuf.at[pl.ds(0, _SEG)], out_hbm.at[pl.ds(out_off, _SEG)])


@functools.partial(
    pl.kernel,
    out_type=jax.ShapeDtypeStruct((_NC, _NPAD, _HH), jnp.float32),
    mesh=_sc_mesh(),
    scratch_types=[
        pltpu.VMEM((_CH2, _K), jnp.int32),
        pltpu.VMEM((_CH2, _K), jnp.int32),
        pltpu.VMEM((_NB, _K, _HH), jnp.float32),
        pltpu.VMEM_SHARED((_NPAD, _HH), jnp.float32),
        [pltpu.SemaphoreType.DMA] * _NB,
        [pltpu.SemaphoreType.DMA] * _NB,
    ],
    compiler_params=pltpu.CompilerParams(use_tc_tiling_on_sc=False),
)
def _sc_msg(y_hbm, r_hbm, c_hbm, out_hbm,
            r_v, c_v, ring, agg_sh, sg, ss):
    cid = lax.axis_index("c")
    sid = lax.axis_index("s")
    base = pl.multiple_of(sid * _SEG, 8)
    ytab = y_hbm.at[cid]
    g = [ring.at[b] for b in range(_NB)]

    @pl.loop(0, _K)
    def _fill_z(i):
        for t in range(_HH // 16):
            ring[0, i, pl.ds(t * 16, 16)] = jnp.zeros((16,), jnp.float32)

    for t in range(4):
        pltpu.sync_copy(g[0], agg_sh.at[pl.ds(base + t * _K, _K)])
    pltpu.sync_copy(g[0].at[pl.ds(0, _SEG - 4 * _K)],
                    agg_sh.at[pl.ds(base + 4 * _K, _SEG - 4 * _K)])
    pltpu.sync_copy(r_hbm.at[sid], r_v)
    pltpu.sync_copy(c_hbm.at[sid], c_v)
    plsc.subcore_barrier()

    def wait_g(b, jj):
        pltpu.make_async_copy(ytab.at[r_v.at[jj]], g[b], sg[b]).wait()

    def wait_s(b, jj):
        pltpu.make_async_copy(g[b], agg_sh.at[c_v.at[jj]], ss[b]).wait()

    _LOOK = _NB // 2
    for b in range(_LOOK):
        pltpu.async_copy(ytab.at[r_v.at[b]], g[b], sg[b])
    for jj in range(_LOOK):
        wait_g(jj, jj)
        pltpu.async_copy(g[jj], agg_sh.at[c_v.at[jj]], ss[jj], add=True)
        bf = jj + _LOOK
        pltpu.async_copy(ytab.at[r_v.at[bf]], g[bf], sg[bf])

    @pl.loop(_LOOK, _CH2 - _LOOK, step=_NB)
    def _chunks(j):
        for b8 in range(_NB):
            jj = j + b8
            b = (b8 + _LOOK) % _NB
            wait_g(b, jj)
            pltpu.async_copy(g[b], agg_sh.at[c_v.at[jj]], ss[b], add=True)
            bf = (b + _LOOK) % _NB
            wait_s(bf, jj - _LOOK)
            pltpu.async_copy(ytab.at[r_v.at[jj + _LOOK]], g[bf], sg[bf])

    for t in range(_LOOK):
        jj = _CH2 - _LOOK + t
        b = jj % _NB
        wait_g(b, jj)
        pltpu.async_copy(g[b], agg_sh.at[c_v.at[jj]], ss[b], add=True)
    for t in range(_NB):
        jj = _CH2 - _NB + t
        wait_s(jj % _NB, jj)

    plsc.subcore_barrier()
    tail = _SEG - 4 * _K
    pltpu.sync_copy(agg_sh.at[pl.ds(base, _K)], g[0])
    for t in range(5):
        cur = g[t % 2]
        nxt = g[(t + 1) % 2]
        if t < 3:
            pltpu.async_copy(agg_sh.at[pl.ds(base + (t + 1) * _K, _K)],
                             nxt, sg[(t + 1) % 2])
        elif t == 3:
            pltpu.async_copy(agg_sh.at[pl.ds(base + 4 * _K, tail)],
                             nxt.at[pl.ds(0, tail)], sg[(t + 1) % 2])
        if t < 4:
            pltpu.sync_copy(cur, out_hbm.at[cid, pl.ds(base + t * _K, _K)])
        else:
            pltpu.sync_copy(cur.at[pl.ds(0, tail)],
                            out_hbm.at[cid, pl.ds(base + 4 * _K, tail)])
        if t < 3:
            pltpu.make_async_copy(agg_sh.at[pl.ds(base, _K)], nxt,
                                  sg[(t + 1) % 2]).wait()
        elif t == 3:
            pltpu.make_async_copy(agg_sh.at[pl.ds(base, tail)],
                                  nxt.at[pl.ds(0, tail)],
                                  sg[(t + 1) % 2]).wait()


def _tc_pre(x, W1, deg3):
    def body(x_ref, w_ref, deg_ref, xw_ref, y_ref, dinv_ref):
        deg = deg_ref[0] + deg_ref[1] + 1.0
        dinv = lax.rsqrt(deg)
        xw = jnp.dot(x_ref[...], w_ref[...], preferred_element_type=jnp.float32,
                     precision=lax.Precision.HIGHEST)
        xw_ref[...] = xw
        y = xw * dinv
        y_ref[0] = y[:, :_HH]
        y_ref[1] = y[:, _HH:]
        dinv_ref[...] = dinv

    return pl.pallas_call(
        body,
        out_shape=[
            jax.ShapeDtypeStruct((_N, _H), jnp.float32),
            jax.ShapeDtypeStruct((_NC, _N, _HH), jnp.float32),
            jax.ShapeDtypeStruct((_N, 1), jnp.float32),
        ],
        compiler_params=pltpu.CompilerParams(
            vmem_limit_bytes=100 * 1024 * 1024),
    )(x, W1, deg3)


def _tc_mid(agg, xw, dinv, b, g, be, W2):
    def body(a_ref, xw_ref, dinv_ref, b_ref, g_ref, be_ref, w_ref,
             xw2_ref, y2_ref):
        dinv = dinv_ref[...]
        edge = jnp.concatenate([a_ref[0, :_N], a_ref[1, :_N]], axis=1)
        t = edge * dinv + xw_ref[...] * (dinv * dinv) + b_ref[...]
        mu = jnp.mean(t, axis=0, keepdims=True)
        var = jnp.mean((t - mu) ** 2, axis=0, keepdims=True)
        h = jnp.maximum((t - mu) * lax.rsqrt(var + _EPS) * g_ref[...]
                        + be_ref[...], 0.0)
        xw2 = jnp.dot(h, w_ref[...], preferred_element_type=jnp.float32,
                     precision=lax.Precision.HIGHEST)
        xw2_ref[...] = xw2
        y2 = xw2 * dinv
        y2_ref[0] = y2[:, :_HH]
        y2_ref[1] = y2[:, _HH:]

    return pl.pallas_call(
        body,
        out_shape=[
            jax.ShapeDtypeStruct((_N, _H), jnp.float32),
            jax.ShapeDtypeStruct((_NC, _N, _HH), jnp.float32),
        ],
        compiler_params=pltpu.CompilerParams(
            vmem_limit_bytes=100 * 1024 * 1024),
    )(agg, xw, dinv, b, g, be, W2)


def _tc_post(agg, xw, dinv, b, g, be, batch_row, Wfc, bfc):
    def body(a_ref, xw_ref, dinv_ref, b_ref, g_ref, be_ref, batch_ref,
             wfc_ref, bfc_ref, out_ref):
        dinv = dinv_ref[...]
        edge = jnp.concatenate([a_ref[0, :_N], a_ref[1, :_N]], axis=1)
        t = edge * dinv + xw_ref[...] * (dinv * dinv) + b_ref[...]
        mu = jnp.mean(t, axis=0, keepdims=True)
        var = jnp.mean((t - mu) ** 2, axis=0, keepdims=True)
        h = jnp.maximum((t - mu) * lax.rsqrt(var + _EPS) * g_ref[...]
                        + be_ref[...], 0.0)
        gi = lax.broadcasted_iota(jnp.int32, (_G, _N), 0)
        oh = (gi == batch_ref[...]).astype(jnp.float32)
        sums = jnp.dot(oh, h, preferred_element_type=jnp.float32,
                     precision=lax.Precision.HIGHEST)
        cnt = jnp.sum(oh, axis=1, keepdims=True)
        pooled = sums / jnp.maximum(cnt, 1.0)
        out_ref[...] = (jnp.dot(pooled, wfc_ref[...],
                                preferred_element_type=jnp.float32,
                     precision=lax.Precision.HIGHEST)
                        + bfc_ref[...])

    return pl.pallas_call(
        body,
        out_shape=jax.ShapeDtypeStruct((_G, _OUT), jnp.float32),
        compiler_params=pltpu.CompilerParams(
            vmem_limit_bytes=100 * 1024 * 1024),
    )(agg, xw, dinv, b, g, be, batch_row, Wfc, bfc)


def kernel(x, edge_index, batch, W1, b1, g1, be1, W2, b2, g2, be2, Wfc, bfc):
    r = edge_index[0]
    c = edge_index[1]
    pad = jnp.arange(_EP - _E, dtype=jnp.int32)
    r3 = jnp.concatenate([r, pad % _N]).reshape(_NS, _CH2, _K)
    c3 = jnp.concatenate(
        [c, _N + pad % (_NPAD - _N)]).reshape(_NS, _CH2, _K)

    deg_parts = _sc_degree(c3)
    deg3 = deg_parts.reshape(_NC, _NPAD)[:, :_N].reshape(_NC, _N, 1)
    xw1, y1s, dinv = _tc_pre(x, W1, deg3)
    agg1 = _sc_msg(y1s, r3, c3)
    xw2, y2s = _tc_mid(agg1, xw1, dinv,
                       b1.reshape(1, _H), g1.reshape(1, _H), be1.reshape(1, _H),
                       W2)
    agg2 = _sc_msg(y2s, r3, c3)
    out = _tc_post(agg2, xw2, dinv,
                   b2.reshape(1, _H), g2.reshape(1, _H), be2.reshape(1, _H),
                   batch.reshape(1, _N), Wfc, bfc.reshape(1, _OUT))
    return out
